# baseline (speedup 1.0000x reference)
import functools

import jax
import jax.numpy as jnp
from jax import lax
from jax.experimental import pallas as pl
from jax.experimental.pallas import tpu as pltpu
from jax.experimental.pallas import tpu_sc as plsc

N_NODES = 10000
N_EDGES = 320000
D = 128
DH = 64
L = 16
NS = 16
NCM = 2
CHUNK = 80
E_PAD = 327680
NCHUNK = E_PAD // (NS * CHUNK)
SG = 16
PCH = 128
NSTAGE = NCHUNK // SG
N_PAD = 10240
ROWS_PER_TILE = N_PAD // NS
QROWS = ROWS_PER_TILE // PCH


def _sc_segment_kernel(x2, u4, v4, zrows, rowids):
    mesh = plsc.VectorSubcoreMesh(
        core_axis_name="c", subcore_axis_name="s", num_cores=NCM)

    @functools.partial(
        pl.kernel,
        out_type=(
            jax.ShapeDtypeStruct((NCM, N_PAD, DH), jnp.float32),
            jax.ShapeDtypeStruct((NCM, NS, N_PAD), jnp.float32),
        ),
        mesh=mesh,
        compiler_params=pltpu.CompilerParams(needs_layout_passes=False, use_tc_tiling_on_sc=False),
        scratch_types=[
            pltpu.VMEM((SG, CHUNK), jnp.int32),
            pltpu.VMEM((SG, CHUNK), jnp.int32),
            pltpu.VMEM((QROWS, PCH), jnp.int32),
            pltpu.VMEM((PCH, DH), jnp.float32),
            pltpu.VMEM((CHUNK, DH), jnp.float32),
            pltpu.VMEM((CHUNK, DH), jnp.float32),
            pltpu.VMEM((CHUNK, DH), jnp.float32),
            pltpu.VMEM((N_PAD,), jnp.float32),
            pltpu.VMEM_SHARED((N_PAD, DH), jnp.float32),
            pltpu.SemaphoreType.DMA,
            pltpu.SemaphoreType.DMA,
            pltpu.SemaphoreType.DMA,
            pltpu.SemaphoreType.DMA,
            pltpu.SemaphoreType.DMA,
            pltpu.SemaphoreType.DMA,
        ],
    )
    def k(x_hbm, u_hbm, v_hbm, zr_hbm, rid_hbm,
          osum_hbm, ocnt_hbm,
          u_v, v_v, rid_v, pub_v, rows_a, rows_b, rows_c, hist_v, acc_s,
          sga, sgb, sgc, ssa, ssb, ssc):
        c = lax.axis_index("c")
        s = lax.axis_index("s")
        base = s * ROWS_PER_TILE
        zero16 = jnp.zeros((L,), jnp.float32)
        one16 = jnp.ones((L,), jnp.float32)
        bufs = (rows_a, rows_b, rows_c)
        gsems = (sga, sgb, sgc)
        ssems = (ssa, ssb, ssc)

        pltpu.sync_copy(zr_hbm, pub_v)
        pltpu.sync_copy(rid_hbm.at[s], rid_v)

        def zslot(i, carry):
            hist_v[pl.ds(i * L, L)] = zero16
            return carry

        lax.fori_loop(0, N_PAD // L, zslot, 0)
        for q in range(QROWS):
            pltpu.sync_copy(pub_v, acc_s.at[rid_v.at[q]])
        plsc.subcore_barrier()

        def stage(st, carry):
            pltpu.sync_copy(u_hbm.at[c, s, pl.ds(st * SG, SG)], u_v)
            pltpu.sync_copy(v_hbm.at[c, s, pl.ds(st * SG, SG)], v_v)

            gathers = [None, None, None]
            scatters = [None, None, None]
            gathers[0] = pltpu.async_copy(
                x_hbm.at[v_v.at[0]], bufs[0], gsems[0])
            gathers[1] = pltpu.async_copy(
                x_hbm.at[v_v.at[1]], bufs[1], gsems[1])
            for j in range(SG):
                b = j % 3
                gathers[b].wait()
                scatters[b] = pltpu.async_copy(
                    bufs[b], acc_s.at[u_v.at[j]], ssems[b], add=True)
                for kk in range(CHUNK // L):
                    idx = u_v[j, pl.ds(kk * L, L)]
                    plsc.addupdate_scatter(hist_v, [idx], one16)
                if j + 2 < SG:
                    nb = (j + 2) % 3
                    if scatters[nb] is not None:
                        scatters[nb].wait()
                    gathers[nb] = pltpu.async_copy(
                        x_hbm.at[v_v.at[j + 2]], bufs[nb], gsems[nb])
            scatters[(SG - 3) % 3].wait()
            scatters[(SG - 2) % 3].wait()
            scatters[(SG - 1) % 3].wait()
            return carry

        lax.fori_loop(0, NSTAGE, stage, 0)

        pltpu.sync_copy(hist_v, ocnt_hbm.at[c, s])
        plsc.subcore_barrier()

        for q in range(QROWS):
            pltpu.async_copy(acc_s.at[rid_v.at[q]], pub_v, sga).wait()
            pltpu.sync_copy(pub_v,
                            osum_hbm.at[c, pl.ds(base + q * PCH, PCH)])

    return k(x2, u4, v4, zrows, rowids)


def _tc_body(x_ref, p_ref, c_ref, w_ref, o_ref):
    cnt = jnp.sum(c_ref[...], axis=0) * 0.5
    cnt = jnp.maximum(cnt, 1.0)
    acc = jnp.dot(x_ref[...], w_ref[0:D, :], preferred_element_type=jnp.float32)
    for h in range(NCM):
        hn = p_ref[h] / cnt
        acc += jnp.dot(hn, w_ref[pl.ds(D + h * DH, DH), :],
                       preferred_element_type=jnp.float32)
    o_ref[...] = jnp.maximum(acc, 0.0)


def _tc_combine(x, parts, cnts, W0):
    blk = 1000
    grid = (N_NODES // blk,)
    return pl.pallas_call(
        _tc_body,
        grid=grid,
        in_specs=[
            pl.BlockSpec((blk, D), lambda i: (i, 0)),
            pl.BlockSpec((NCM, blk, DH), lambda i: (0, i, 0)),
            pl.BlockSpec((NCM * NS, blk, 1), lambda i: (0, i, 0)),
            pl.BlockSpec((2 * D, D), lambda i: (0, 0)),
        ],
        out_specs=pl.BlockSpec((blk, D), lambda i: (i, 0)),
        out_shape=jax.ShapeDtypeStruct((N_NODES, D), jnp.float32),
    )(x, parts, cnts, W0)


def kernel(x, edge_index, W0):
    u = edge_index[0]
    v = edge_index[1]
    pad = E_PAD - N_EDGES
    u_pad = jnp.concatenate([u, jnp.full((pad,), N_NODES, jnp.int32)])
    v_pad = jnp.concatenate([v, jnp.zeros((pad,), jnp.int32)])
    u4 = jnp.broadcast_to(u_pad.reshape(1, NS, NCHUNK, CHUNK),
                          (NCM, NS, NCHUNK, CHUNK))
    v4 = jnp.stack([2 * v_pad, 2 * v_pad + 1]).reshape(NCM, NS, NCHUNK, CHUNK)
    x2 = x.reshape(2 * N_NODES, DH)
    zrows = jnp.zeros((PCH, DH), jnp.float32)
    rowids = jnp.arange(N_PAD, dtype=jnp.int32).reshape(NS, QROWS, PCH)
    parts, cnts = _sc_segment_kernel(x2, u4, v4, zrows, rowids)
    cnts_col = cnts.reshape(NCM * NS, N_PAD, 1)
    return _tc_combine(x, parts, cnts_col, W0)


# speedup vs baseline: 1.5732x; 1.5732x over previous
import functools

import jax
import jax.numpy as jnp
from jax import lax
from jax.experimental import pallas as pl
from jax.experimental.pallas import tpu as pltpu
from jax.experimental.pallas import tpu_sc as plsc

N_NODES = 10000
N_EDGES = 320000
D = 128
DH = 64
L = 16
NS = 16
NCM = 2
CHUNK = 80
E_PAD = N_EDGES
NCHUNK = E_PAD // (NS * CHUNK)
SG = 10
PCH = 128
NSTAGE = NCHUNK // SG
N_PAD = 10240
ROWS_PER_TILE = N_PAD // NS
QROWS = ROWS_PER_TILE // PCH


def _sc_segment_kernel(x2, u4, v4, zrows, rowids):
    mesh = plsc.VectorSubcoreMesh(
        core_axis_name="c", subcore_axis_name="s", num_cores=NCM)

    @functools.partial(
        pl.kernel,
        out_type=(
            jax.ShapeDtypeStruct((NCM, N_PAD, DH), jnp.float32),
            jax.ShapeDtypeStruct((NCM, NS, N_PAD), jnp.float32),
        ),
        mesh=mesh,
        compiler_params=pltpu.CompilerParams(needs_layout_passes=False, use_tc_tiling_on_sc=False),
        scratch_types=[
            pltpu.VMEM((SG, CHUNK), jnp.int32),
            pltpu.VMEM((SG, CHUNK), jnp.int32),
            pltpu.VMEM((QROWS, PCH), jnp.int32),
            pltpu.VMEM((PCH, DH), jnp.float32),
            pltpu.VMEM((CHUNK, DH), jnp.float32),
            pltpu.VMEM((CHUNK, DH), jnp.float32),
            pltpu.VMEM((CHUNK, DH), jnp.float32),
            pltpu.VMEM((N_PAD,), jnp.float32),
            pltpu.VMEM_SHARED((N_PAD, DH), jnp.float32),
            pltpu.SemaphoreType.DMA,
            pltpu.SemaphoreType.DMA,
            pltpu.SemaphoreType.DMA,
            pltpu.SemaphoreType.DMA,
            pltpu.SemaphoreType.DMA,
            pltpu.SemaphoreType.DMA,
        ],
    )
    def k(x_hbm, u_hbm, v_hbm, zr_hbm, rid_hbm,
          osum_hbm, ocnt_hbm,
          u_v, v_v, rid_v, pub_v, rows_a, rows_b, rows_c, hist_v, acc_s,
          sga, sgb, sgc, ssa, ssb, ssc):
        c = lax.axis_index("c")
        s = lax.axis_index("s")
        base = s * ROWS_PER_TILE
        zero16 = jnp.zeros((L,), jnp.float32)
        one16 = jnp.ones((L,), jnp.float32)
        bufs = (rows_a, rows_b, rows_c)
        gsems = (sga, sgb, sgc)
        ssems = (ssa, ssb, ssc)

        pltpu.sync_copy(zr_hbm, pub_v)
        pltpu.sync_copy(rid_hbm.at[s], rid_v)

        def zslot(i, carry):
            hist_v[pl.ds(i * L, L)] = zero16
            return carry

        lax.fori_loop(0, N_PAD // L, zslot, 0)
        for q in range(QROWS):
            pltpu.sync_copy(pub_v, acc_s.at[rid_v.at[q]])
        plsc.subcore_barrier()

        def stage(st, carry):
            pltpu.sync_copy(u_hbm.at[s, pl.ds(st * SG, SG)], u_v)
            pltpu.sync_copy(v_hbm.at[c, s, pl.ds(st * SG, SG)], v_v)

            gathers = [None, None, None]
            scatters = [None, None, None]
            gathers[0] = pltpu.async_copy(
                x_hbm.at[v_v.at[0]], bufs[0], gsems[0])
            gathers[1] = pltpu.async_copy(
                x_hbm.at[v_v.at[1]], bufs[1], gsems[1])
            for j in range(SG):
                b = j % 3
                gathers[b].wait()
                scatters[b] = pltpu.async_copy(
                    bufs[b], acc_s.at[u_v.at[j]], ssems[b], add=True)
                for kk in range(CHUNK // L):
                    idx = u_v[j, pl.ds(kk * L, L)]
                    plsc.addupdate_scatter(hist_v, [idx], one16)
                if j + 2 < SG:
                    nb = (j + 2) % 3
                    if scatters[nb] is not None:
                        scatters[nb].wait()
                    gathers[nb] = pltpu.async_copy(
                        x_hbm.at[v_v.at[j + 2]], bufs[nb], gsems[nb])
            scatters[(SG - 3) % 3].wait()
            scatters[(SG - 2) % 3].wait()
            scatters[(SG - 1) % 3].wait()
            return carry

        lax.fori_loop(0, NSTAGE, stage, 0)

        pltpu.sync_copy(hist_v, ocnt_hbm.at[c, s])
        plsc.subcore_barrier()

        for q in range(QROWS):
            pltpu.async_copy(acc_s.at[rid_v.at[q]], pub_v, sga).wait()
            pltpu.sync_copy(pub_v,
                            osum_hbm.at[c, pl.ds(base + q * PCH, PCH)])

    return k(x2, u4, v4, zrows, rowids)


def _tc_body(x_ref, p_ref, c_ref, w_ref, o_ref):
    cnt = jnp.sum(c_ref[...], axis=0) * 0.5
    cnt = jnp.maximum(cnt, 1.0)
    acc = jnp.dot(x_ref[...], w_ref[0:D, :], preferred_element_type=jnp.float32)
    for h in range(NCM):
        hn = p_ref[h] / cnt
        acc += jnp.dot(hn, w_ref[pl.ds(D + h * DH, DH), :],
                       preferred_element_type=jnp.float32)
    o_ref[...] = jnp.maximum(acc, 0.0)


def _tc_combine(x, parts, cnts, W0):
    blk = 1000
    grid = (N_NODES // blk,)
    return pl.pallas_call(
        _tc_body,
        grid=grid,
        in_specs=[
            pl.BlockSpec((blk, D), lambda i: (i, 0)),
            pl.BlockSpec((NCM, blk, DH), lambda i: (0, i, 0)),
            pl.BlockSpec((NCM * NS, blk, 1), lambda i: (0, i, 0)),
            pl.BlockSpec((2 * D, D), lambda i: (0, 0)),
        ],
        out_specs=pl.BlockSpec((blk, D), lambda i: (i, 0)),
        out_shape=jax.ShapeDtypeStruct((N_NODES, D), jnp.float32),
    )(x, parts, cnts, W0)


def kernel(x, edge_index, W0):
    u = edge_index[0]
    v = edge_index[1]
    u4 = u.reshape(NS, NCHUNK, CHUNK)
    v4 = jnp.stack([2 * v, 2 * v + 1]).reshape(NCM, NS, NCHUNK, CHUNK)
    x2 = x.reshape(2 * N_NODES, DH)
    zrows = jnp.zeros((PCH, DH), jnp.float32)
    rowids = jnp.arange(N_PAD, dtype=jnp.int32).reshape(NS, QROWS, PCH)
    parts, cnts = _sc_segment_kernel(x2, u4, v4, zrows, rowids)
    cnts_col = cnts.reshape(NCM * NS, N_PAD, 1)
    return _tc_combine(x, parts, cnts_col, W0)
